# baseline (speedup 1.0000x reference)
import functools

import jax
import jax.numpy as jnp
from jax import lax
from jax.experimental import pallas as pl
from jax.experimental.pallas import tpu as pltpu
from jax.experimental.pallas import tpu_sc as plsc

N = 10000
E = 320000
D = 128

NC, NS = 2, 16
NW = NC * NS
CH = 128
CPW0 = 160
CPW1 = 0
SCH0 = 32
NBUF = 2
NCHUNK = NS * CPW0
EPAD = NCHUNK * CH
ACC_ROWS = 10240
PAD_DST = N
RPW = ACC_ROWS // NS

MM_BLK = 1000


def _matmul(x, w):
    def body(x_ref, w_ref, o_ref):
        o_ref[...] = jnp.dot(x_ref[...], w_ref[...],
                             preferred_element_type=jnp.float32)

    return pl.pallas_call(
        body,
        grid=(N // MM_BLK,),
        in_specs=[
            pl.BlockSpec((MM_BLK, D), lambda i: (i, 0)),
            pl.BlockSpec((D, D), lambda i: (0, 0)),
        ],
        out_specs=pl.BlockSpec((MM_BLK, D), lambda i: (i, 0)),
        out_shape=jax.ShapeDtypeStruct((N, D), jnp.float32),
    )(x, w)


def _sc_scatter(h0, h1, h2, src, dst, zeros):
    mesh = plsc.VectorSubcoreMesh(core_axis_name="c", subcore_axis_name="s")

    @functools.partial(
        pl.kernel,
        out_type=jax.ShapeDtypeStruct((3, ACC_ROWS, D), jnp.float32),
        mesh=mesh,
        scratch_types=[
            pltpu.VMEM((SCH0, CH), jnp.int32),
            pltpu.VMEM((SCH0, CH), jnp.int32),
            [pltpu.VMEM((CH, D), jnp.float32) for _ in range(NBUF)],
            pltpu.VMEM_SHARED((ACC_ROWS, D), jnp.float32),
            [pltpu.SemaphoreType.DMA for _ in range(NBUF)],
        ],
    )
    def k(h0_hbm, h1_hbm, h2_hbm, src_hbm, dst_hbm, z_hbm, p_hbm,
          srcv, dstv, rows, acc, sems):
        c = lax.axis_index("c")
        s = lax.axis_index("s")
        hs = (h0_hbm, h1_hbm, h2_hbm)

        def edge_phase(g, base, cpw, sch):
            h = hs[g]

            def gather(ci, b):
                pltpu.async_copy(h.at[srcv.at[ci]], rows[b], sems[b])

            def gather_wait(ci, b):
                pltpu.make_async_copy(h.at[srcv.at[ci]], rows[b], sems[b]).wait()

            for strip in range(cpw // sch):
                sbase = base + strip * sch
                pltpu.sync_copy(src_hbm.at[g, pl.ds(sbase, sch)],
                                srcv.at[pl.ds(0, sch)])
                pltpu.sync_copy(dst_hbm.at[g, pl.ds(sbase, sch)],
                                dstv.at[pl.ds(0, sch)])

                for b in range(NBUF):
                    gather(b, b)

                def body(o, carry):
                    for b in range(NBUF):
                        ci = o * NBUF + b
                        gather_wait(ci, b)
                        pltpu.sync_copy(rows[b], acc.at[dstv.at[ci]], add=True)
                        gather(ci + NBUF, b)
                    return carry

                lax.fori_loop(0, sch // NBUF - 1, body, 0)
                for b in range(NBUF):
                    ci = sch - NBUF + b
                    gather_wait(ci, b)
                    pltpu.sync_copy(rows[b], acc.at[dstv.at[ci]], add=True)

        @pl.when(c == 0)
        def _():
            pltpu.sync_copy(z_hbm.at[pl.ds(s * RPW, RPW)],
                            acc.at[pl.ds(s * RPW, RPW)])
            plsc.subcore_barrier()

            for g in range(3):
                edge_phase(g, s * CPW0, CPW0, SCH0)
                plsc.subcore_barrier()
                pltpu.sync_copy(acc.at[pl.ds(s * RPW, RPW)],
                                p_hbm.at[g, pl.ds(s * RPW, RPW)])
                if g < 2:
                    pltpu.sync_copy(z_hbm.at[pl.ds(s * RPW, RPW)],
                                    acc.at[pl.ds(s * RPW, RPW)])
                plsc.subcore_barrier()

    return k(h0, h1, h2, src, dst, zeros)


def _epilogue(p, g, x, b):
    def body(p_ref, x_ref, b_ref, o_ref):
        agg = p_ref[0] + b_ref[...]
        o_ref[...] = jnp.maximum(agg, 0.0) + x_ref[...]

    return pl.pallas_call(
        body,
        grid=(N // MM_BLK,),
        in_specs=[
            pl.BlockSpec((1, MM_BLK, D), lambda i, g=g: (g, i, 0)),
            pl.BlockSpec((MM_BLK, D), lambda i: (i, 0)),
            pl.BlockSpec((1, D), lambda i: (0, 0)),
        ],
        out_specs=pl.BlockSpec((MM_BLK, D), lambda i: (i, 0)),
        out_shape=jax.ShapeDtypeStruct((N, D), jnp.float32),
    )(p, x, b)


def _prep_edges(edge_index):
    src = edge_index[0].astype(jnp.int32)
    dst = edge_index[1].astype(jnp.int32)
    pad_dst = PAD_DST + jnp.arange(EPAD - E, dtype=jnp.int32) % (ACC_ROWS - N)
    src = jnp.concatenate([src, jnp.zeros((EPAD - E,), jnp.int32)])
    dst = jnp.concatenate([dst, pad_dst])
    return src.reshape(NCHUNK, CH), dst.reshape(NCHUNK, CH)


def kernel(h_mf_new, h_bp_new, h_cc_new, mf_edge_index, bp_edge_index,
           cc_edge_index, W_mf, b_mf, W_bp, b_bp, W_cc, b_cc):
    xs = (h_mf_new, h_bp_new, h_cc_new)
    hs = tuple(_matmul(x, w) for x, w in zip(xs, (W_mf, W_bp, W_cc)))

    se, de = zip(*(_prep_edges(e) for e in
                   (mf_edge_index, bp_edge_index, cc_edge_index)))
    src = jnp.stack(se)
    dst = jnp.stack(de)
    zeros = jnp.zeros((ACC_ROWS, D), jnp.float32)

    p = _sc_scatter(hs[0], hs[1], hs[2], src, dst, zeros)

    bs = (b_mf, b_bp, b_cc)
    outs = tuple(_epilogue(p, g, xs[g], bs[g].reshape(1, D)) for g in range(3))
    return outs


# speedup vs baseline: 1.1842x; 1.1842x over previous
import functools

import jax
import jax.numpy as jnp
from jax import lax
from jax.experimental import pallas as pl
from jax.experimental.pallas import tpu as pltpu
from jax.experimental.pallas import tpu_sc as plsc

N = 10000
E = 320000
D = 128

NC, NS = 2, 16
NW = NC * NS
CH = 128
CPW = 80
SCH = 16
NBUF = 2
NCHUNK = NW * CPW
EPAD = NCHUNK * CH
ACC_ROWS = 10240
PAD_DST = N
RPW = ACC_ROWS // NS

MM_BLK = 1000


def _matmul(x, w):
    def body(x_ref, w_ref, o_ref):
        o_ref[...] = jnp.dot(x_ref[...], w_ref[...],
                             preferred_element_type=jnp.float32)

    return pl.pallas_call(
        body,
        grid=(N // MM_BLK,),
        in_specs=[
            pl.BlockSpec((MM_BLK, D), lambda i: (i, 0)),
            pl.BlockSpec((D, D), lambda i: (0, 0)),
        ],
        out_specs=pl.BlockSpec((MM_BLK, D), lambda i: (i, 0)),
        out_shape=jax.ShapeDtypeStruct((N, D), jnp.float32),
    )(x, w)


def _sc_scatter(h0, h1, h2, src, dst):
    mesh = plsc.VectorSubcoreMesh(core_axis_name="c", subcore_axis_name="s")

    @functools.partial(
        pl.kernel,
        out_type=jax.ShapeDtypeStruct((3, NC, ACC_ROWS, D), jnp.float32),
        mesh=mesh,
        scratch_types=[
            pltpu.VMEM((SCH, CH), jnp.int32),
            pltpu.VMEM((SCH, CH), jnp.int32),
            [pltpu.VMEM((CH, D), jnp.float32) for _ in range(NBUF)],
            pltpu.VMEM_SHARED((ACC_ROWS, D), jnp.float32),
            [pltpu.SemaphoreType.DMA for _ in range(NBUF)],
        ],
    )
    def k(h0_hbm, h1_hbm, h2_hbm, src_hbm, dst_hbm, p_hbm,
          srcv, dstv, rows, acc, sems):
        c = lax.axis_index("c")
        s = lax.axis_index("s")
        wid = c * NS + s
        hs = (h0_hbm, h1_hbm, h2_hbm)

        def fill_zero():
            def fb(r, carry):
                for j in range(D // 16):
                    rows[0][r, pl.ds(j * 16, 16)] = jnp.zeros((16,), jnp.float32)
                return carry
            lax.fori_loop(0, CH, fb, 0)

        def zero_slice():
            for j in range(RPW // CH):
                pltpu.sync_copy(rows[0], acc.at[pl.ds(s * RPW + j * CH, CH)])

        def edge_phase(g):
            h = hs[g]

            def gather(ci, b):
                pltpu.async_copy(h.at[srcv.at[ci]], rows[b], sems[b])

            def gather_wait(ci, b):
                pltpu.make_async_copy(h.at[srcv.at[ci]], rows[b], sems[b]).wait()

            for strip in range(CPW // SCH):
                sbase = wid * CPW + strip * SCH
                pltpu.sync_copy(src_hbm.at[g, pl.ds(sbase, SCH)], srcv)
                pltpu.sync_copy(dst_hbm.at[g, pl.ds(sbase, SCH)], dstv)

                for b in range(NBUF):
                    gather(b, b)

                def body(o, carry):
                    for b in range(NBUF):
                        ci = o * NBUF + b
                        gather_wait(ci, b)
                        pltpu.sync_copy(rows[b], acc.at[dstv.at[ci]], add=True)
                        gather(ci + NBUF, b)
                    return carry

                lax.fori_loop(0, SCH // NBUF - 1, body, 0)
                for b in range(NBUF):
                    ci = SCH - NBUF + b
                    gather_wait(ci, b)
                    pltpu.sync_copy(rows[b], acc.at[dstv.at[ci]], add=True)

        fill_zero()
        zero_slice()
        plsc.subcore_barrier()

        for g in range(3):
            edge_phase(g)
            plsc.subcore_barrier()
            pltpu.sync_copy(acc.at[pl.ds(s * RPW, RPW)],
                            p_hbm.at[g, c, pl.ds(s * RPW, RPW)])
            if g < 2:
                fill_zero()
                zero_slice()
            plsc.subcore_barrier()

    return k(h0, h1, h2, src, dst)


def _epilogue(p, g, x, b):
    def body(p0_ref, p1_ref, x_ref, b_ref, o_ref):
        agg = p0_ref[0, 0] + p1_ref[0, 0] + b_ref[...]
        o_ref[...] = jnp.maximum(agg, 0.0) + x_ref[...]

    return pl.pallas_call(
        body,
        grid=(N // MM_BLK,),
        in_specs=[
            pl.BlockSpec((1, 1, MM_BLK, D), lambda i, g=g: (g, 0, i, 0)),
            pl.BlockSpec((1, 1, MM_BLK, D), lambda i, g=g: (g, 1, i, 0)),
            pl.BlockSpec((MM_BLK, D), lambda i: (i, 0)),
            pl.BlockSpec((1, D), lambda i: (0, 0)),
        ],
        out_specs=pl.BlockSpec((MM_BLK, D), lambda i: (i, 0)),
        out_shape=jax.ShapeDtypeStruct((N, D), jnp.float32),
    )(p, p, x, b)


def _prep_edges(edge_index):
    src = edge_index[0].astype(jnp.int32)
    dst = edge_index[1].astype(jnp.int32)
    pad_dst = PAD_DST + jnp.arange(EPAD - E, dtype=jnp.int32) % (ACC_ROWS - N)
    src = jnp.concatenate([src, jnp.zeros((EPAD - E,), jnp.int32)])
    dst = jnp.concatenate([dst, pad_dst])
    return src.reshape(NCHUNK, CH), dst.reshape(NCHUNK, CH)


def kernel(h_mf_new, h_bp_new, h_cc_new, mf_edge_index, bp_edge_index,
           cc_edge_index, W_mf, b_mf, W_bp, b_bp, W_cc, b_cc):
    xs = (h_mf_new, h_bp_new, h_cc_new)
    hs = tuple(_matmul(x, w) for x, w in zip(xs, (W_mf, W_bp, W_cc)))

    se, de = zip(*(_prep_edges(e) for e in
                   (mf_edge_index, bp_edge_index, cc_edge_index)))
    src = jnp.stack(se)
    dst = jnp.stack(de)

    p = _sc_scatter(hs[0], hs[1], hs[2], src, dst)

    bs = (b_mf, b_bp, b_cc)
    outs = tuple(_epilogue(p, g, xs[g], bs[g].reshape(1, D)) for g in range(3))
    return outs


# speedup vs baseline: 3.7882x; 3.1989x over previous
import functools

import jax
import jax.numpy as jnp
from jax import lax
from jax.experimental import pallas as pl
from jax.experimental.pallas import tpu as pltpu
from jax.experimental.pallas import tpu_sc as plsc

N = 10000
E = 320000
D = 128

NC, NS = 2, 16
NW = NC * NS
CH = 128
CPW = 80
SCH = 16
NBUF = 2
NCHUNK = NW * CPW
EPAD = NCHUNK * CH
ACC_ROWS = 10240
PAD_DST = N
RPW = ACC_ROWS // NS

MM_BLK = 1000


def _matmul(x, w):
    def body(x_ref, w_ref, o_ref):
        o_ref[...] = jnp.dot(x_ref[...], w_ref[...],
                             preferred_element_type=jnp.float32)

    return pl.pallas_call(
        body,
        grid=(N // MM_BLK,),
        in_specs=[
            pl.BlockSpec((MM_BLK, D), lambda i: (i, 0)),
            pl.BlockSpec((D, D), lambda i: (0, 0)),
        ],
        out_specs=pl.BlockSpec((MM_BLK, D), lambda i: (i, 0)),
        out_shape=jax.ShapeDtypeStruct((N, D), jnp.float32),
    )(x, w)


def _sc_scatter(h0, h1, h2, src, dst):
    mesh = plsc.VectorSubcoreMesh(core_axis_name="c", subcore_axis_name="s")

    @functools.partial(
        pl.kernel,
        out_type=jax.ShapeDtypeStruct((3, NC, ACC_ROWS, D), jnp.float32),
        mesh=mesh,
        scratch_types=[
            pltpu.VMEM((SCH, CH), jnp.int32),
            pltpu.VMEM((SCH, CH), jnp.int32),
            [pltpu.VMEM((CH, D), jnp.float32) for _ in range(NBUF)],
            pltpu.VMEM_SHARED((ACC_ROWS, D), jnp.float32),
            [pltpu.SemaphoreType.DMA for _ in range(NBUF)],
        ],
    )
    def k(h0_hbm, h1_hbm, h2_hbm, src_hbm, dst_hbm, p_hbm,
          srcv, dstv, rows, acc, sems):
        c = lax.axis_index("c")
        s = lax.axis_index("s")
        wid = c * NS + s
        hs = (h0_hbm, h1_hbm, h2_hbm)

        def fill_zero():
            def fb(r, carry):
                for j in range(D // 16):
                    rows[0][r, pl.ds(j * 16, 16)] = jnp.zeros((16,), jnp.float32)
                return carry
            lax.fori_loop(0, CH, fb, 0)

        def zero_slice():
            for j in range(RPW // CH):
                pltpu.sync_copy(rows[0], acc.at[pl.ds(s * RPW + j * CH, CH)])

        def edge_phase(g):
            h = hs[g]

            def gather(ci, b):
                pltpu.async_copy(h.at[srcv.at[ci]], rows[b], sems[b])

            def gather_wait(ci, b):
                pltpu.make_async_copy(h.at[srcv.at[ci]], rows[b], sems[b]).wait()

            for strip in range(CPW // SCH):
                sbase = wid * CPW + strip * SCH
                pltpu.sync_copy(src_hbm.at[g, pl.ds(sbase, SCH)], srcv)
                pltpu.sync_copy(dst_hbm.at[g, pl.ds(sbase, SCH)], dstv)

                for b in range(NBUF):
                    gather(b, b)

                def body(o, carry):
                    for b in range(NBUF):
                        ci = o * NBUF + b
                        gather_wait(ci, b)
                        pltpu.sync_copy(rows[b], acc.at[dstv.at[ci]], add=True)
                        gather(ci + NBUF, b)
                    return carry

                lax.fori_loop(0, SCH // NBUF - 1, body, 0)
                for b in range(NBUF):
                    ci = SCH - NBUF + b
                    gather_wait(ci, b)
                    pltpu.sync_copy(rows[b], acc.at[dstv.at[ci]], add=True)

        fill_zero()
        zero_slice()
        plsc.subcore_barrier()

        for g in range(3):
            edge_phase(g)
            plsc.subcore_barrier()
            pltpu.sync_copy(acc.at[pl.ds(s * RPW, RPW)],
                            p_hbm.at[g, c, pl.ds(s * RPW, RPW)])
            if g < 2:
                fill_zero()
                zero_slice()
            plsc.subcore_barrier()

    return k(h0, h1, h2, src, dst)


def _epilogue(p, g, x, b):
    def body(p0_ref, p1_ref, x_ref, b_ref, o_ref):
        agg = p0_ref[0, 0] + p1_ref[0, 0] + b_ref[...]
        o_ref[...] = jnp.maximum(agg, 0.0) + x_ref[...]

    return pl.pallas_call(
        body,
        grid=(N // MM_BLK,),
        in_specs=[
            pl.BlockSpec((1, 1, MM_BLK, D), lambda i, g=g: (g, 0, i, 0)),
            pl.BlockSpec((1, 1, MM_BLK, D), lambda i, g=g: (g, 1, i, 0)),
            pl.BlockSpec((MM_BLK, D), lambda i: (i, 0)),
            pl.BlockSpec((1, D), lambda i: (0, 0)),
        ],
        out_specs=pl.BlockSpec((MM_BLK, D), lambda i: (i, 0)),
        out_shape=jax.ShapeDtypeStruct((N, D), jnp.float32),
    )(p, p, x, b)


def _prep_edges(edge_index):
    src = edge_index[0].astype(jnp.int32)
    dst = edge_index[1].astype(jnp.int32)
    pad = jnp.arange(EPAD - E, dtype=jnp.int32)
    pad_src = (pad * 197) % N
    pad_dst = PAD_DST + pad % (ACC_ROWS - N)
    src = jnp.concatenate([src, pad_src])
    dst = jnp.concatenate([dst, pad_dst])
    return src.reshape(NCHUNK, CH), dst.reshape(NCHUNK, CH)


def kernel(h_mf_new, h_bp_new, h_cc_new, mf_edge_index, bp_edge_index,
           cc_edge_index, W_mf, b_mf, W_bp, b_bp, W_cc, b_cc):
    xs = (h_mf_new, h_bp_new, h_cc_new)
    hs = tuple(_matmul(x, w) for x, w in zip(xs, (W_mf, W_bp, W_cc)))

    se, de = zip(*(_prep_edges(e) for e in
                   (mf_edge_index, bp_edge_index, cc_edge_index)))
    src = jnp.stack(se)
    dst = jnp.stack(de)

    p = _sc_scatter(hs[0], hs[1], hs[2], src, dst)

    bs = (b_mf, b_bp, b_cc)
    outs = tuple(_epilogue(p, g, xs[g], bs[g].reshape(1, D)) for g in range(3))
    return outs
